# baseline (speedup 1.0000x reference)
import functools

import jax
import jax.numpy as jnp
from jax import lax
from jax.experimental import pallas as pl
from jax.experimental.pallas import tpu as pltpu
from jax.experimental.pallas import tpu_sc as plsc

_NC, _NS = 2, 16
_NW = _NC * _NS
_R = 16
_L = 16


def _sc_body(B, S, D, x_hbm, t_hbm, pos_hbm, out_hbm, bufx, buft, idx_v, sem):
    cid = lax.axis_index("c")
    sid = lax.axis_index("s")
    wid = sid * _NC + cid
    rows_per_w = S // _NW
    nchunks = rows_per_w // _R
    seqbase = wid * rows_per_w

    def chunk(ci, carry):
        rows = seqbase + ci * _R
        pltpu.sync_copy(pos_hbm.at[pl.ds(rows, _R)], idx_v)
        pltpu.async_copy(t_hbm.at[idx_v], buft, sem).wait()

        def batch(b, c2):
            pltpu.sync_copy(x_hbm.at[b, pl.ds(rows, _R)], bufx)

            def add_blk(j, c3):
                base = j * _L
                for r in range(_R):
                    bufx[r, pl.ds(base, _L)] = (
                        bufx[r, pl.ds(base, _L)] + buft[r, pl.ds(base, _L)]
                    )
                return c3

            lax.fori_loop(0, D // _L, add_blk, 0)
            pltpu.sync_copy(bufx, out_hbm.at[b, pl.ds(rows, _R)])
            return c2

        return lax.fori_loop(0, B, batch, carry)

    lax.fori_loop(0, nchunks, chunk, 0)


def kernel(x, table, offset=0):
    B, S, D = x.shape
    pos = jnp.asarray(offset, jnp.int32) + lax.iota(jnp.int32, S)
    mesh = plsc.VectorSubcoreMesh(core_axis_name="c", subcore_axis_name="s")
    body = functools.partial(_sc_body, B, S, D)
    f = pl.kernel(
        body,
        out_type=jax.ShapeDtypeStruct((B, S, D), x.dtype),
        mesh=mesh,
        scratch_types=[
            pltpu.VMEM((_R, D), jnp.float32),
            pltpu.VMEM((_R, D), jnp.float32),
            pltpu.VMEM((_R,), jnp.int32),
            pltpu.SemaphoreType.DMA,
        ],
    )
    return f(x, table, pos)


# speedup vs baseline: 1.9749x; 1.9749x over previous
import functools

import jax
import jax.numpy as jnp
from jax import lax
from jax.experimental import pallas as pl
from jax.experimental.pallas import tpu as pltpu
from jax.experimental.pallas import tpu_sc as plsc

_NC, _NS = 2, 16
_NW = _NC * _NS
_R = 16
_L = 16


def _sc_body(B, S, D, x_hbm, t_hbm, pos_hbm, out_hbm,
             bx0, bx1, bx2, bx3, bt0, bt1, idx_all,
             sl0, sl1, sl2, sl3, ss0, ss1, ss2, ss3, st0, st1):
    cid = lax.axis_index("c")
    sid = lax.axis_index("s")
    wid = sid * _NC + cid
    rows_per_w = S // _NW
    nchunks = rows_per_w // _R
    seqbase = wid * rows_per_w

    bufx = (bx0, bx1, bx2, bx3)
    buft = (bt0, bt1)
    sld = (sl0, sl1, sl2, sl3)
    sst = (ss0, ss1, ss2, ss3)
    stb = (st0, st1)

    def x_slice(ci, b):
        return x_hbm.at[b, pl.ds(seqbase + ci * _R, _R)]

    def out_slice(ci, b):
        return out_hbm.at[b, pl.ds(seqbase + ci * _R, _R)]

    def start_gather(ci, tp):
        idx = idx_all.at[pl.ds(ci * _R, _R)]
        pltpu.async_copy(t_hbm.at[idx], buft[tp], stb[tp])

    pltpu.sync_copy(pos_hbm.at[pl.ds(seqbase, rows_per_w)], idx_all)

    start_gather(0, 0)
    pltpu.async_copy(x_slice(0, 0), bufx[0], sld[0])
    pltpu.async_copy(x_slice(0, 1), bufx[1], sld[1])

    def two_chunks(i, carry):
        for half in range(2):
            ci = 2 * i + half
            tp = half
            pltpu.make_async_copy(
                t_hbm.at[idx_all.at[pl.ds(0, _R)]], buft[tp], stb[tp]).wait()
            @pl.when(ci + 1 < nchunks)
            def _():
                start_gather(ci + 1, 1 - tp)

            for b in range(B):
                pltpu.make_async_copy(x_slice(0, 0), bufx[b], sld[b]).wait()

                def add_blk(j, c3):
                    base = j * _L
                    for r in range(_R):
                        bufx[b][r, pl.ds(base, _L)] = (
                            bufx[b][r, pl.ds(base, _L)]
                            + buft[tp][r, pl.ds(base, _L)]
                        )
                    return c3

                lax.fori_loop(0, D // _L, add_blk, 0)
                pltpu.async_copy(bufx[b], out_slice(ci, b), sst[b])

                nb = (b + 2) % B
                if b < 2:
                    @pl.when(ci >= 1)
                    def _():
                        pltpu.make_async_copy(
                            bufx[nb], out_slice(0, 0), sst[nb]).wait()
                    pltpu.async_copy(x_slice(ci, nb), bufx[nb], sld[nb])
                else:
                    @pl.when(ci + 1 < nchunks)
                    def _():
                        pltpu.make_async_copy(
                            bufx[nb], out_slice(0, 0), sst[nb]).wait()
                        pltpu.async_copy(x_slice(ci + 1, nb), bufx[nb], sld[nb])
        return carry

    lax.fori_loop(0, nchunks // 2, two_chunks, 0)

    for b in range(B):
        pltpu.make_async_copy(bufx[b], out_slice(0, 0), sst[b]).wait()


def kernel(x, table, offset=0):
    B, S, D = x.shape
    pos = jnp.asarray(offset, jnp.int32) + lax.iota(jnp.int32, S)
    mesh = plsc.VectorSubcoreMesh(core_axis_name="c", subcore_axis_name="s")
    body = functools.partial(_sc_body, B, S, D)
    f = pl.kernel(
        body,
        out_type=jax.ShapeDtypeStruct((B, S, D), x.dtype),
        mesh=mesh,
        scratch_types=(
            [pltpu.VMEM((_R, D), jnp.float32) for _ in range(4)]
            + [pltpu.VMEM((_R, D), jnp.float32) for _ in range(2)]
            + [pltpu.VMEM((S // _NW,), jnp.int32)]
            + [pltpu.SemaphoreType.DMA for _ in range(10)]
        ),
    )
    return f(x, table, pos)


# speedup vs baseline: 2.4365x; 1.2337x over previous
import functools

import jax
import jax.numpy as jnp
from jax import lax
from jax.experimental import pallas as pl
from jax.experimental.pallas import tpu as pltpu
from jax.experimental.pallas import tpu_sc as plsc

_NC, _NS = 2, 16
_NW = _NC * _NS
_R = 8
_L = 16


def _sc_body(B, S, D, x_hbm, t_hbm, pos_hbm, out_hbm, *refs):
    bufx = (refs[0:4], refs[4:8])
    buft = refs[8:10]
    idx_all = refs[10]
    sld = (refs[11:15], refs[15:19])
    sst = (refs[19:23], refs[23:27])
    stb = refs[27:29]

    cid = lax.axis_index("c")
    sid = lax.axis_index("s")
    wid = sid * _NC + cid
    rows_per_w = S // _NW
    nchunks = rows_per_w // _R
    seqbase = wid * rows_per_w

    def x_slice(ci, b):
        return x_hbm.at[b, pl.ds(seqbase + ci * _R, _R)]

    def out_slice(ci, b):
        return out_hbm.at[b, pl.ds(seqbase + ci * _R, _R)]

    def start_gather(ci, p):
        idx = idx_all.at[pl.ds(ci * _R, _R)]
        pltpu.async_copy(t_hbm.at[idx], buft[p], stb[p])

    pltpu.sync_copy(pos_hbm.at[pl.ds(seqbase, rows_per_w)], idx_all)

    start_gather(0, 0)
    for b in range(B):
        pltpu.async_copy(x_slice(0, b), bufx[0][b], sld[0][b])

    def two_chunks(i, carry):
        for p in range(2):
            ci = 2 * i + p
            q = 1 - p
            pltpu.make_async_copy(
                t_hbm.at[idx_all.at[pl.ds(0, _R)]], buft[p], stb[p]).wait()
            @pl.when(ci + 1 < nchunks)
            def _():
                start_gather(ci + 1, q)

            for b in range(B):
                pltpu.make_async_copy(
                    x_slice(0, 0), bufx[p][b], sld[p][b]).wait()
            @pl.when(ci + 1 < nchunks)
            def _():
                for b in range(B):
                    @pl.when(ci >= 1)
                    def _():
                        pltpu.make_async_copy(
                            bufx[q][b], out_slice(0, 0), sst[q][b]).wait()
                    pltpu.async_copy(x_slice(ci + 1, b), bufx[q][b], sld[q][b])

            @plsc.parallel_loop(0, D // _L, unroll=2)
            def _(j):
                base = j * _L
                for r in range(_R):
                    vt = buft[p][r, pl.ds(base, _L)]
                    for b in range(B):
                        bufx[p][b][r, pl.ds(base, _L)] = (
                            bufx[p][b][r, pl.ds(base, _L)] + vt
                        )

            for b in range(B):
                pltpu.async_copy(bufx[p][b], out_slice(ci, b), sst[p][b])
        return carry

    lax.fori_loop(0, nchunks // 2, two_chunks, 0)

    for p in range(2):
        for b in range(B):
            pltpu.make_async_copy(bufx[p][b], out_slice(0, 0), sst[p][b]).wait()


def kernel(x, table, offset=0):
    B, S, D = x.shape
    pos = jnp.asarray(offset, jnp.int32) + lax.iota(jnp.int32, S)
    mesh = plsc.VectorSubcoreMesh(core_axis_name="c", subcore_axis_name="s")
    body = functools.partial(_sc_body, B, S, D)
    f = pl.kernel(
        body,
        out_type=jax.ShapeDtypeStruct((B, S, D), x.dtype),
        mesh=mesh,
        scratch_types=(
            [pltpu.VMEM((_R, D), jnp.float32) for _ in range(8)]
            + [pltpu.VMEM((_R, D), jnp.float32) for _ in range(2)]
            + [pltpu.VMEM((S // _NW,), jnp.int32)]
            + [pltpu.SemaphoreType.DMA for _ in range(18)]
        ),
    )
    return f(x, table, pos)


# speedup vs baseline: 2.4379x; 1.0006x over previous
import functools

import jax
import jax.numpy as jnp
from jax import lax
from jax.experimental import pallas as pl
from jax.experimental.pallas import tpu as pltpu
from jax.experimental.pallas import tpu_sc as plsc

_NC, _NS = 2, 16
_NW = _NC * _NS
_R = 8
_L = 16


def _sc_body(B, S, D, x_hbm, t_hbm, pos_hbm, out_hbm, *refs):
    bufx = (refs[0:4], refs[4:8])
    buft = refs[8:10]
    idx_all = refs[10]
    sld = (refs[11:15], refs[15:19])
    sst = (refs[19:23], refs[23:27])
    stb = refs[27:29]

    cid = lax.axis_index("c")
    sid = lax.axis_index("s")
    wid = sid * _NC + cid
    rows_per_w = S // _NW
    nchunks = rows_per_w // _R
    seqbase = wid * rows_per_w

    def x_slice(ci, b):
        return x_hbm.at[b, pl.ds(seqbase + ci * _R, _R)]

    def out_slice(ci, b):
        return out_hbm.at[b, pl.ds(seqbase + ci * _R, _R)]

    def start_gather(ci, p):
        idx = idx_all.at[pl.ds(ci * _R, _R)]
        pltpu.async_copy(t_hbm.at[idx], buft[p], stb[p])

    pltpu.sync_copy(pos_hbm.at[pl.ds(seqbase, rows_per_w)], idx_all)

    start_gather(0, 0)
    for b in range(B):
        pltpu.async_copy(x_slice(0, b), bufx[0][b], sld[0][b])

    def two_chunks(i, carry):
        for p in range(2):
            ci = 2 * i + p
            q = 1 - p
            pltpu.make_async_copy(
                t_hbm.at[idx_all.at[pl.ds(0, _R)]], buft[p], stb[p]).wait()
            @pl.when(ci + 1 < nchunks)
            def _():
                start_gather(ci + 1, q)

            for b in range(B):
                pltpu.make_async_copy(
                    x_slice(0, 0), bufx[p][b], sld[p][b]).wait()
            @pl.when(ci + 1 < nchunks)
            def _():
                for b in range(B):
                    @pl.when(ci >= 1)
                    def _():
                        pltpu.make_async_copy(
                            bufx[q][b], out_slice(0, 0), sst[q][b]).wait()
                    pltpu.async_copy(x_slice(ci + 1, b), bufx[q][b], sld[q][b])

            @plsc.parallel_loop(0, D // _L, unroll=4)
            def _(j):
                base = j * _L
                for r in range(_R):
                    vt = buft[p][r, pl.ds(base, _L)]
                    for b in range(B):
                        bufx[p][b][r, pl.ds(base, _L)] = (
                            bufx[p][b][r, pl.ds(base, _L)] + vt
                        )

            for b in range(B):
                pltpu.async_copy(bufx[p][b], out_slice(ci, b), sst[p][b])
        return carry

    lax.fori_loop(0, nchunks // 2, two_chunks, 0)

    for p in range(2):
        for b in range(B):
            pltpu.make_async_copy(bufx[p][b], out_slice(0, 0), sst[p][b]).wait()


def kernel(x, table, offset=0):
    B, S, D = x.shape
    pos = jnp.asarray(offset, jnp.int32) + lax.iota(jnp.int32, S)
    mesh = plsc.VectorSubcoreMesh(core_axis_name="c", subcore_axis_name="s")
    body = functools.partial(_sc_body, B, S, D)
    f = pl.kernel(
        body,
        out_type=jax.ShapeDtypeStruct((B, S, D), x.dtype),
        mesh=mesh,
        scratch_types=(
            [pltpu.VMEM((_R, D), jnp.float32) for _ in range(8)]
            + [pltpu.VMEM((_R, D), jnp.float32) for _ in range(2)]
            + [pltpu.VMEM((S // _NW,), jnp.int32)]
            + [pltpu.SemaphoreType.DMA for _ in range(18)]
        ),
    )
    return f(x, table, pos)
